# baseline (speedup 1.0000x reference)
import functools

import jax
import jax.numpy as jnp
from jax import lax
from jax.experimental import pallas as pl
from jax.experimental.pallas import tpu as pltpu
from jax.experimental.pallas import tpu_sc as plsc

_G = 128
_C = 256
_NSL = _C // 16

_SC_ROWS = 40960
_SC_OFF = 50000 - _SC_ROWS
_NW = 32
_RPW = _SC_ROWS // _NW
_CH = 128
_NCH = _RPW // _CH

_NSTEPS = 10
_RB = _SC_OFF // _NSTEPS
_RG = 50000 // _NSTEPS


def _sc_body(b_hbm, ids_hbm, w_hbm, sums_hbm, cnts_hbm,
             buf0, buf1, idsv, w1v, sums_v, cnts_v, acc_v, sem0, sem1):
    wid = lax.axis_index("s") * 2 + lax.axis_index("c")
    base = _SC_OFF + wid * _RPW

    pltpu.sync_copy(w_hbm.at[:, pl.ds(0, _C)], w1v)
    pltpu.sync_copy(ids_hbm.at[pl.ds(base, _RPW)], idsv.at[pl.ds(0, _RPW)])
    pltpu.async_copy(b_hbm.at[pl.ds(base, _CH), :], buf0, sem0)
    pltpu.async_copy(b_hbm.at[pl.ds(base + _CH, _CH), :], buf1, sem1)

    zf = jnp.zeros((16,), jnp.float32)
    zi = jnp.zeros((16,), jnp.int32)
    for gi in range(_G):
        sums_v[gi, pl.ds(0, 16)] = zf
        cnts_v[gi, pl.ds(0, 16)] = zi
    for j in range(_NSL):
        acc_v[j, pl.ds(0, 16)] = zf

    def flush(g, cnt):
        @pl.when(g >= 0)
        def _():
            v = acc_v[0, pl.ds(0, 16)] * w1v[0, pl.ds(0, 16)]
            for j in range(1, _NSL):
                v = v + acc_v[j, pl.ds(0, 16)] * w1v[0, pl.ds(j * 16, 16)]
            sums_v[g, pl.ds(0, 16)] = v
            cnts_v[g, pl.ds(0, 16)] = jnp.full((16,), cnt, jnp.int32)

    def make_group_body(buf, choff):
        def group_body(gq, carry):
            g = carry[0]
            rloc = gq * 16
            first = idsv[pl.ds(choff + rloc, 16)][0]
            last = idsv[pl.ds(choff + rloc + 15, 16)][0]
            same = (first == g) & (last == g)

            def fast(ops):
                t = [buf[rloc, pl.ds(j * 16, 16)] for j in range(_NSL)]
                for k in range(1, 16):
                    t = [a + buf[rloc + k, pl.ds(j * 16, 16)]
                         for j, a in enumerate(t)]
                for j in range(_NSL):
                    acc_v[j, pl.ds(0, 16)] += t[j]
                return (ops[0], ops[1] + 16)

            def slow(ops):
                def row_body(k, c2):
                    g1, cnt1 = c2
                    gr = idsv[pl.ds(choff + rloc + k, 16)][0]
                    changed = gr != g1
                    @pl.when(changed)
                    def _():
                        flush(g1, cnt1)
                        for j in range(_NSL):
                            acc_v[j, pl.ds(0, 16)] = zf
                    cnt1 = jnp.where(changed, 0, cnt1)
                    for j in range(_NSL):
                        acc_v[j, pl.ds(0, 16)] += buf[rloc + k,
                                                      pl.ds(j * 16, 16)]
                    return (gr, cnt1 + 1)
                return lax.fori_loop(0, 16, row_body, ops)

            return lax.cond(same, fast, slow, carry)
        return group_body

    def pair_body(p, carry):
        for bsel, (bufb, semb) in enumerate(((buf0, sem0), (buf1, sem1))):
            ch = p * 2 + bsel
            choff = ch * _CH
            pltpu.make_async_copy(b_hbm.at[pl.ds(0, _CH), :], bufb, semb).wait()
            carry = lax.fori_loop(0, _CH // 16,
                                  make_group_body(bufb, choff), carry)
            @pl.when(ch + 2 < _NCH)
            def _():
                nstart = base + (ch + 2) * _CH
                pltpu.async_copy(b_hbm.at[pl.ds(nstart, _CH), :], bufb, semb)
        return carry

    carry = lax.fori_loop(0, _NCH // 2, pair_body,
                          (jnp.int32(-1), jnp.int32(0)))
    flush(carry[0], carry[1])

    pltpu.sync_copy(sums_v, sums_hbm.at[wid])
    pltpu.sync_copy(cnts_v, cnts_hbm.at[wid])


def _sc_partials(B_z, ids_b, W):
    fn = functools.partial(
        pl.kernel,
        mesh=plsc.VectorSubcoreMesh(core_axis_name="c", subcore_axis_name="s"),
        out_type=[jax.ShapeDtypeStruct((_NW, _G, 16), jnp.float32),
                  jax.ShapeDtypeStruct((_NW, _G, 16), jnp.int32)],
        scratch_types=[pltpu.VMEM((_CH, _C), jnp.float32),
                       pltpu.VMEM((_CH, _C), jnp.float32),
                       pltpu.VMEM((_RPW + 16,), jnp.int32),
                       pltpu.VMEM((1, _C), jnp.float32),
                       pltpu.VMEM((_G, 16), jnp.float32),
                       pltpu.VMEM((_G, 16), jnp.int32),
                       pltpu.VMEM((_NSL, 16), jnp.float32),
                       pltpu.SemaphoreType.DMA,
                       pltpu.SemaphoreType.DMA],
    )(_sc_body)
    return fn(B_z, ids_b, W)


def _tc_main_body(ib_ref, ig_ref, bsh_ref, g_ref, w_ref, accb_ref, accg_ref):
    i = pl.program_id(0)

    @pl.when(i == 0)
    def _init():
        accb_ref[...] = jnp.zeros_like(accb_ref)
        accg_ref[...] = jnp.zeros_like(accg_ref)

    w1 = w_ref[0, :_C]
    w2 = w_ref[0, _C:]
    sv_b = jnp.sum(bsh_ref[...] * w1[None, :], axis=1, keepdims=True)
    sv_g = jnp.sum(g_ref[...] * w2[None, :], axis=1, keepdims=True)
    svc_b = jnp.concatenate([sv_b, jnp.ones_like(sv_b)], axis=1)
    svc_g = jnp.concatenate([sv_g, jnp.ones_like(sv_g)], axis=1)
    ids_b = ib_ref[0]
    ids_g = ig_ref[0]
    seg_b = lax.broadcasted_iota(jnp.int32, (_G, _RB), 0)
    seg_g = lax.broadcasted_iota(jnp.int32, (_G, _RG), 0)
    oh_b = (seg_b == ids_b).astype(jnp.float32)
    oh_g = (seg_g == ids_g).astype(jnp.float32)
    dn = (((1,), (0,)), ((), ()))
    accb_ref[...] += lax.dot_general(
        oh_b, svc_b, dn, preferred_element_type=jnp.float32)
    accg_ref[...] += lax.dot_general(
        oh_g, svc_g, dn, preferred_element_type=jnp.float32)


def _tc_main(ids_b_sh, ids_g, B_z, G_z, W):
    return pl.pallas_call(
        _tc_main_body,
        grid=(_NSTEPS,),
        in_specs=[
            pl.BlockSpec((1, 1, _RB), lambda i: (i, 0, 0)),
            pl.BlockSpec((1, 1, _RG), lambda i: (i, 0, 0)),
            pl.BlockSpec((_RB, _C), lambda i: (i, 0)),
            pl.BlockSpec((_RG, _C), lambda i: (i, 0)),
            pl.BlockSpec((1, 2 * _C), lambda i: (0, 0)),
        ],
        out_specs=[pl.BlockSpec((_G, 2), lambda i: (0, 0)),
                   pl.BlockSpec((_G, 2), lambda i: (0, 0))],
        out_shape=[jax.ShapeDtypeStruct((_G, 2), jnp.float32),
                   jax.ShapeDtypeStruct((_G, 2), jnp.float32)],
        compiler_params=pltpu.CompilerParams(
            dimension_semantics=("arbitrary",)),
    )(ids_b_sh, ids_g, B_z, G_z, W)


def _epi_body(scs_ref, scc_ref, accb_ref, accg_ref, bias_ref, out_ref):
    scs = jnp.sum(scs_ref[...], axis=(0, 2))
    scc = jnp.sum(scc_ref[...], axis=(0, 2)).astype(jnp.float32) / 16.0
    bsum = accb_ref[:, 0] + scs
    bcnt = accb_ref[:, 1] + scc
    res = (bsum / jnp.maximum(bcnt, 1.0)
           + accg_ref[:, 0] / jnp.maximum(accg_ref[:, 1], 1.0)
           + bias_ref[0, 0])
    out_ref[...] = res[:, None]


def _epilogue(sc_sums, sc_cnts, accb, accg, bias):
    return pl.pallas_call(
        _epi_body,
        out_shape=jax.ShapeDtypeStruct((_G, 1), jnp.float32),
    )(sc_sums, sc_cnts, accb, accg, bias)


def kernel(B_z, G_z, x_b_batch, x_g_batch, W, b):
    ids_b = x_b_batch.astype(jnp.int32)
    ids_g = x_g_batch.astype(jnp.int32)
    sc_sums, sc_cnts = _sc_partials(B_z, ids_b, W)
    ids_b_sh = ids_b[:_SC_OFF].reshape(_NSTEPS, 1, _RB)
    ids_g_r = ids_g.reshape(_NSTEPS, 1, _RG)
    accb, accg = _tc_main(ids_b_sh, ids_g_r, B_z, G_z, W)
    return _epilogue(sc_sums, sc_cnts, accb, accg, b.reshape(1, 1))


# speedup vs baseline: 1.3996x; 1.3996x over previous
import functools

import jax
import jax.numpy as jnp
from jax import lax
from jax.experimental import pallas as pl
from jax.experimental.pallas import tpu as pltpu
from jax.experimental.pallas import tpu_sc as plsc

_G = 128
_C = 256
_NSL = _C // 16

_SC_ROWS = 40960
_SC_OFF = 50000 - _SC_ROWS
_NW = 32
_RPW = _SC_ROWS // _NW
_CH = 128
_NCH = _RPW // _CH

_NSTEPS = 10
_RB = _SC_OFF // _NSTEPS
_RG = 50000 // _NSTEPS


def _sc_body(b_hbm, ids_hbm, w_hbm, sums_hbm, cnts_hbm,
             buf0, buf1, idsv, w1v, sums_v, cnts_v, acc_v, sem0, sem1):
    wid = lax.axis_index("s") * 2 + lax.axis_index("c")
    base = _SC_OFF + wid * _RPW

    pltpu.sync_copy(w_hbm.at[:, pl.ds(0, _C)], w1v)
    pltpu.sync_copy(ids_hbm.at[pl.ds(base, _RPW)], idsv.at[pl.ds(0, _RPW)])
    pltpu.async_copy(b_hbm.at[pl.ds(base, _CH), :], buf0, sem0)
    pltpu.async_copy(b_hbm.at[pl.ds(base + _CH, _CH), :], buf1, sem1)

    zf = jnp.zeros((16,), jnp.float32)
    zi = jnp.zeros((16,), jnp.int32)
    for gi in range(_G):
        sums_v[gi, pl.ds(0, 16)] = zf
        cnts_v[gi, pl.ds(0, 16)] = zi
    for j in range(_NSL):
        acc_v[j, pl.ds(0, 16)] = zf

    def flush(g, cnt):
        @pl.when(g >= 0)
        def _():
            v = acc_v[0, pl.ds(0, 16)] * w1v[0, pl.ds(0, 16)]
            for j in range(1, _NSL):
                v = v + acc_v[j, pl.ds(0, 16)] * w1v[0, pl.ds(j * 16, 16)]
            sums_v[g, pl.ds(0, 16)] = v
            cnts_v[g, pl.ds(0, 16)] = jnp.full((16,), cnt, jnp.int32)

    def make_group_body(buf, choff):
        def group_body(gq, carry):
            g = carry[0]
            rloc = gq * 16
            first = idsv[pl.ds(choff + rloc, 16)][0]
            last = idsv[pl.ds(choff + rloc + 15, 16)][0]
            same = (first == g) & (last == g)

            def fast(ops):
                for j in range(_NSL):
                    dsj = pl.ds(j * 16, 16)
                    a0 = buf[rloc, dsj] + buf[rloc + 1, dsj]
                    a1 = buf[rloc + 2, dsj] + buf[rloc + 3, dsj]
                    a2 = buf[rloc + 4, dsj] + buf[rloc + 5, dsj]
                    a3 = buf[rloc + 6, dsj] + buf[rloc + 7, dsj]
                    for k in range(8, 16, 4):
                        a0 = a0 + buf[rloc + k, dsj]
                        a1 = a1 + buf[rloc + k + 1, dsj]
                        a2 = a2 + buf[rloc + k + 2, dsj]
                        a3 = a3 + buf[rloc + k + 3, dsj]
                    acc_v[j, pl.ds(0, 16)] += (a0 + a1) + (a2 + a3)
                return (ops[0], ops[1] + 16)

            def slow(ops):
                def row_body(k, c2):
                    g1, cnt1 = c2
                    gr = idsv[pl.ds(choff + rloc + k, 16)][0]
                    changed = gr != g1
                    @pl.when(changed)
                    def _():
                        flush(g1, cnt1)
                        for j in range(_NSL):
                            acc_v[j, pl.ds(0, 16)] = zf
                    cnt1 = jnp.where(changed, 0, cnt1)
                    for j in range(_NSL):
                        acc_v[j, pl.ds(0, 16)] += buf[rloc + k,
                                                      pl.ds(j * 16, 16)]
                    return (gr, cnt1 + 1)
                return lax.fori_loop(0, 16, row_body, ops)

            return lax.cond(same, fast, slow, carry)
        return group_body

    def pair_body(p, carry):
        for bsel, (bufb, semb) in enumerate(((buf0, sem0), (buf1, sem1))):
            ch = p * 2 + bsel
            choff = ch * _CH
            pltpu.make_async_copy(b_hbm.at[pl.ds(0, _CH), :], bufb, semb).wait()
            carry = lax.fori_loop(0, _CH // 16,
                                  make_group_body(bufb, choff), carry)
            @pl.when(ch + 2 < _NCH)
            def _():
                nstart = base + (ch + 2) * _CH
                pltpu.async_copy(b_hbm.at[pl.ds(nstart, _CH), :], bufb, semb)
        return carry

    carry = lax.fori_loop(0, _NCH // 2, pair_body,
                          (jnp.int32(-1), jnp.int32(0)))
    flush(carry[0], carry[1])

    pltpu.sync_copy(sums_v, sums_hbm.at[wid])
    pltpu.sync_copy(cnts_v, cnts_hbm.at[wid])


def _sc_partials(B_z, ids_b, W):
    fn = functools.partial(
        pl.kernel,
        mesh=plsc.VectorSubcoreMesh(core_axis_name="c", subcore_axis_name="s"),
        out_type=[jax.ShapeDtypeStruct((_NW, _G, 16), jnp.float32),
                  jax.ShapeDtypeStruct((_NW, _G, 16), jnp.int32)],
        scratch_types=[pltpu.VMEM((_CH, _C), jnp.float32),
                       pltpu.VMEM((_CH, _C), jnp.float32),
                       pltpu.VMEM((_RPW + 16,), jnp.int32),
                       pltpu.VMEM((1, _C), jnp.float32),
                       pltpu.VMEM((_G, 16), jnp.float32),
                       pltpu.VMEM((_G, 16), jnp.int32),
                       pltpu.VMEM((_NSL, 16), jnp.float32),
                       pltpu.SemaphoreType.DMA,
                       pltpu.SemaphoreType.DMA],
    )(_sc_body)
    return fn(B_z, ids_b, W)


def _tc_main_body(ib_ref, ig_ref, bsh_ref, g_ref, w_ref, accb_ref, accg_ref):
    i = pl.program_id(0)

    @pl.when(i == 0)
    def _init():
        accb_ref[...] = jnp.zeros_like(accb_ref)
        accg_ref[...] = jnp.zeros_like(accg_ref)

    w1 = w_ref[0, :_C]
    w2 = w_ref[0, _C:]
    sv_b = jnp.sum(bsh_ref[...] * w1[None, :], axis=1, keepdims=True)
    sv_g = jnp.sum(g_ref[...] * w2[None, :], axis=1, keepdims=True)
    svc_b = jnp.concatenate([sv_b, jnp.ones_like(sv_b)], axis=1)
    svc_g = jnp.concatenate([sv_g, jnp.ones_like(sv_g)], axis=1)
    ids_b = ib_ref[0]
    ids_g = ig_ref[0]
    seg_b = lax.broadcasted_iota(jnp.int32, (_G, _RB), 0)
    seg_g = lax.broadcasted_iota(jnp.int32, (_G, _RG), 0)
    oh_b = (seg_b == ids_b).astype(jnp.float32)
    oh_g = (seg_g == ids_g).astype(jnp.float32)
    dn = (((1,), (0,)), ((), ()))
    accb_ref[...] += lax.dot_general(
        oh_b, svc_b, dn, preferred_element_type=jnp.float32)
    accg_ref[...] += lax.dot_general(
        oh_g, svc_g, dn, preferred_element_type=jnp.float32)


def _tc_main(ids_b_sh, ids_g, B_z, G_z, W):
    return pl.pallas_call(
        _tc_main_body,
        grid=(_NSTEPS,),
        in_specs=[
            pl.BlockSpec((1, 1, _RB), lambda i: (i, 0, 0)),
            pl.BlockSpec((1, 1, _RG), lambda i: (i, 0, 0)),
            pl.BlockSpec((_RB, _C), lambda i: (i, 0)),
            pl.BlockSpec((_RG, _C), lambda i: (i, 0)),
            pl.BlockSpec((1, 2 * _C), lambda i: (0, 0)),
        ],
        out_specs=[pl.BlockSpec((_G, 2), lambda i: (0, 0)),
                   pl.BlockSpec((_G, 2), lambda i: (0, 0))],
        out_shape=[jax.ShapeDtypeStruct((_G, 2), jnp.float32),
                   jax.ShapeDtypeStruct((_G, 2), jnp.float32)],
        compiler_params=pltpu.CompilerParams(
            dimension_semantics=("arbitrary",)),
    )(ids_b_sh, ids_g, B_z, G_z, W)


def _epi_body(scs_ref, scc_ref, accb_ref, accg_ref, bias_ref, out_ref):
    scs = jnp.sum(scs_ref[...], axis=(0, 2))
    scc = jnp.sum(scc_ref[...], axis=(0, 2)).astype(jnp.float32) / 16.0
    bsum = accb_ref[:, 0] + scs
    bcnt = accb_ref[:, 1] + scc
    res = (bsum / jnp.maximum(bcnt, 1.0)
           + accg_ref[:, 0] / jnp.maximum(accg_ref[:, 1], 1.0)
           + bias_ref[0, 0])
    out_ref[...] = res[:, None]


def _epilogue(sc_sums, sc_cnts, accb, accg, bias):
    return pl.pallas_call(
        _epi_body,
        out_shape=jax.ShapeDtypeStruct((_G, 1), jnp.float32),
    )(sc_sums, sc_cnts, accb, accg, bias)


def kernel(B_z, G_z, x_b_batch, x_g_batch, W, b):
    ids_b = x_b_batch.astype(jnp.int32)
    ids_g = x_g_batch.astype(jnp.int32)
    sc_sums, sc_cnts = _sc_partials(B_z, ids_b, W)
    ids_b_sh = ids_b[:_SC_OFF].reshape(_NSTEPS, 1, _RB)
    ids_g_r = ids_g.reshape(_NSTEPS, 1, _RG)
    accb, accg = _tc_main(ids_b_sh, ids_g_r, B_z, G_z, W)
    return _epilogue(sc_sums, sc_cnts, accb, accg, b.reshape(1, 1))


# speedup vs baseline: 1.5969x; 1.1409x over previous
import functools

import jax
import jax.numpy as jnp
from jax import lax
from jax.experimental import pallas as pl
from jax.experimental.pallas import tpu as pltpu
from jax.experimental.pallas import tpu_sc as plsc

_G = 128
_C = 256
_NSL = _C // 16

_SC_ROWS = 25600
_SC_OFF = 50000 - _SC_ROWS
_NW = 32
_RPW = _SC_ROWS // _NW
_CH = 80
_NCH = _RPW // _CH

_NSTEPS = 10
_RB = _SC_OFF // _NSTEPS
_RG = 50000 // _NSTEPS


def _sc_body(b_hbm, ids_hbm, w_hbm, sums_hbm, cnts_hbm,
             buf0, buf1, idsv, w1v, sums_v, cnts_v, acc_v, sem0, sem1):
    wid = lax.axis_index("s") * 2 + lax.axis_index("c")
    base = _SC_OFF + wid * _RPW

    pltpu.sync_copy(w_hbm.at[:, pl.ds(0, _C)], w1v)
    pltpu.sync_copy(ids_hbm.at[pl.ds(base, _RPW)], idsv.at[pl.ds(0, _RPW)])
    pltpu.async_copy(b_hbm.at[pl.ds(base, _CH), :], buf0, sem0)
    pltpu.async_copy(b_hbm.at[pl.ds(base + _CH, _CH), :], buf1, sem1)

    zf = jnp.zeros((16,), jnp.float32)
    zi = jnp.zeros((16,), jnp.int32)
    for gi in range(_G):
        sums_v[gi, pl.ds(0, 16)] = zf
        cnts_v[gi, pl.ds(0, 16)] = zi
    for j in range(_NSL):
        acc_v[j, pl.ds(0, 16)] = zf

    def flush(g, cnt):
        @pl.when(g >= 0)
        def _():
            v = acc_v[0, pl.ds(0, 16)] * w1v[0, pl.ds(0, 16)]
            for j in range(1, _NSL):
                v = v + acc_v[j, pl.ds(0, 16)] * w1v[0, pl.ds(j * 16, 16)]
            sums_v[g, pl.ds(0, 16)] = v
            cnts_v[g, pl.ds(0, 16)] = jnp.full((16,), cnt, jnp.int32)

    def make_group_body(buf, choff):
        def group_body(gq, carry):
            g = carry[0]
            rloc = gq * 16
            first = idsv[pl.ds(choff + rloc, 16)][0]
            last = idsv[pl.ds(choff + rloc + 15, 16)][0]
            same = (first == g) & (last == g)

            def fast(ops):
                for j in range(_NSL):
                    dsj = pl.ds(j * 16, 16)
                    a0 = buf[rloc, dsj] + buf[rloc + 1, dsj]
                    a1 = buf[rloc + 2, dsj] + buf[rloc + 3, dsj]
                    a2 = buf[rloc + 4, dsj] + buf[rloc + 5, dsj]
                    a3 = buf[rloc + 6, dsj] + buf[rloc + 7, dsj]
                    for k in range(8, 16, 4):
                        a0 = a0 + buf[rloc + k, dsj]
                        a1 = a1 + buf[rloc + k + 1, dsj]
                        a2 = a2 + buf[rloc + k + 2, dsj]
                        a3 = a3 + buf[rloc + k + 3, dsj]
                    acc_v[j, pl.ds(0, 16)] += (a0 + a1) + (a2 + a3)
                return (ops[0], ops[1] + 16)

            def slow(ops):
                def row_body(k, c2):
                    g1, cnt1 = c2
                    gr = idsv[pl.ds(choff + rloc + k, 16)][0]
                    changed = gr != g1
                    @pl.when(changed)
                    def _():
                        flush(g1, cnt1)
                        for j in range(_NSL):
                            acc_v[j, pl.ds(0, 16)] = zf
                    cnt1 = jnp.where(changed, 0, cnt1)
                    for j in range(_NSL):
                        acc_v[j, pl.ds(0, 16)] += buf[rloc + k,
                                                      pl.ds(j * 16, 16)]
                    return (gr, cnt1 + 1)
                return lax.fori_loop(0, 16, row_body, ops)

            return lax.cond(same, fast, slow, carry)
        return group_body

    def pair_body(p, carry):
        for bsel, (bufb, semb) in enumerate(((buf0, sem0), (buf1, sem1))):
            ch = p * 2 + bsel
            choff = ch * _CH
            pltpu.make_async_copy(b_hbm.at[pl.ds(0, _CH), :], bufb, semb).wait()
            carry = lax.fori_loop(0, _CH // 16,
                                  make_group_body(bufb, choff), carry)
            @pl.when(ch + 2 < _NCH)
            def _():
                nstart = base + (ch + 2) * _CH
                pltpu.async_copy(b_hbm.at[pl.ds(nstart, _CH), :], bufb, semb)
        return carry

    carry = lax.fori_loop(0, _NCH // 2, pair_body,
                          (jnp.int32(-1), jnp.int32(0)))
    flush(carry[0], carry[1])

    pltpu.sync_copy(sums_v, sums_hbm.at[wid])
    pltpu.sync_copy(cnts_v, cnts_hbm.at[wid])


def _sc_partials(B_z, ids_b, W):
    fn = functools.partial(
        pl.kernel,
        mesh=plsc.VectorSubcoreMesh(core_axis_name="c", subcore_axis_name="s"),
        out_type=[jax.ShapeDtypeStruct((_NW, _G, 16), jnp.float32),
                  jax.ShapeDtypeStruct((_NW, _G, 16), jnp.int32)],
        scratch_types=[pltpu.VMEM((_CH, _C), jnp.float32),
                       pltpu.VMEM((_CH, _C), jnp.float32),
                       pltpu.VMEM((_RPW + 16,), jnp.int32),
                       pltpu.VMEM((1, _C), jnp.float32),
                       pltpu.VMEM((_G, 16), jnp.float32),
                       pltpu.VMEM((_G, 16), jnp.int32),
                       pltpu.VMEM((_NSL, 16), jnp.float32),
                       pltpu.SemaphoreType.DMA,
                       pltpu.SemaphoreType.DMA],
    )(_sc_body)
    return fn(B_z, ids_b, W)


def _tc_main_body(ib_ref, ig_ref, bsh_ref, g_ref, w_ref, accb_ref, accg_ref):
    i = pl.program_id(0)

    @pl.when(i == 0)
    def _init():
        accb_ref[...] = jnp.zeros_like(accb_ref)
        accg_ref[...] = jnp.zeros_like(accg_ref)

    w1 = w_ref[0, :_C]
    w2 = w_ref[0, _C:]
    sv_b = jnp.sum(bsh_ref[...] * w1[None, :], axis=1, keepdims=True)
    sv_g = jnp.sum(g_ref[...] * w2[None, :], axis=1, keepdims=True)
    svc_b = jnp.concatenate([sv_b, jnp.ones_like(sv_b)], axis=1)
    svc_g = jnp.concatenate([sv_g, jnp.ones_like(sv_g)], axis=1)
    ids_b = ib_ref[0]
    ids_g = ig_ref[0]
    seg_b = lax.broadcasted_iota(jnp.int32, (_G, _RB), 0)
    seg_g = lax.broadcasted_iota(jnp.int32, (_G, _RG), 0)
    oh_b = (seg_b == ids_b).astype(jnp.float32)
    oh_g = (seg_g == ids_g).astype(jnp.float32)
    dn = (((1,), (0,)), ((), ()))
    accb_ref[...] += lax.dot_general(
        oh_b, svc_b, dn, preferred_element_type=jnp.float32)
    accg_ref[...] += lax.dot_general(
        oh_g, svc_g, dn, preferred_element_type=jnp.float32)


def _tc_main(ids_b_sh, ids_g, B_z, G_z, W):
    return pl.pallas_call(
        _tc_main_body,
        grid=(_NSTEPS,),
        in_specs=[
            pl.BlockSpec((1, 1, _RB), lambda i: (i, 0, 0)),
            pl.BlockSpec((1, 1, _RG), lambda i: (i, 0, 0)),
            pl.BlockSpec((_RB, _C), lambda i: (i, 0)),
            pl.BlockSpec((_RG, _C), lambda i: (i, 0)),
            pl.BlockSpec((1, 2 * _C), lambda i: (0, 0)),
        ],
        out_specs=[pl.BlockSpec((_G, 2), lambda i: (0, 0)),
                   pl.BlockSpec((_G, 2), lambda i: (0, 0))],
        out_shape=[jax.ShapeDtypeStruct((_G, 2), jnp.float32),
                   jax.ShapeDtypeStruct((_G, 2), jnp.float32)],
        compiler_params=pltpu.CompilerParams(
            dimension_semantics=("arbitrary",)),
    )(ids_b_sh, ids_g, B_z, G_z, W)


def _epi_body(scs_ref, scc_ref, accb_ref, accg_ref, bias_ref, out_ref):
    scs = jnp.sum(scs_ref[...], axis=(0, 2))
    scc = jnp.sum(scc_ref[...], axis=(0, 2)).astype(jnp.float32) / 16.0
    bsum = accb_ref[:, 0] + scs
    bcnt = accb_ref[:, 1] + scc
    res = (bsum / jnp.maximum(bcnt, 1.0)
           + accg_ref[:, 0] / jnp.maximum(accg_ref[:, 1], 1.0)
           + bias_ref[0, 0])
    out_ref[...] = res[:, None]


def _epilogue(sc_sums, sc_cnts, accb, accg, bias):
    return pl.pallas_call(
        _epi_body,
        out_shape=jax.ShapeDtypeStruct((_G, 1), jnp.float32),
    )(sc_sums, sc_cnts, accb, accg, bias)


def kernel(B_z, G_z, x_b_batch, x_g_batch, W, b):
    ids_b = x_b_batch.astype(jnp.int32)
    ids_g = x_g_batch.astype(jnp.int32)
    sc_sums, sc_cnts = _sc_partials(B_z, ids_b, W)
    ids_b_sh = ids_b[:_SC_OFF].reshape(_NSTEPS, 1, _RB)
    ids_g_r = ids_g.reshape(_NSTEPS, 1, _RG)
    accb, accg = _tc_main(ids_b_sh, ids_g_r, B_z, G_z, W)
    return _epilogue(sc_sums, sc_cnts, accb, accg, b.reshape(1, 1))


# speedup vs baseline: 1.6168x; 1.0125x over previous
import functools

import jax
import jax.numpy as jnp
from jax import lax
from jax.experimental import pallas as pl
from jax.experimental.pallas import tpu as pltpu
from jax.experimental.pallas import tpu_sc as plsc

_G = 128
_C = 256
_NSL = _C // 16

_SC_ROWS = 15360
_SC_OFF = 50000 - _SC_ROWS
_NW = 32
_RPW = _SC_ROWS // _NW
_CH = 80
_NCH = _RPW // _CH

_NSTEPS = 10
_RB = _SC_OFF // _NSTEPS
_RG = 50000 // _NSTEPS


def _sc_body(b_hbm, ids_hbm, w_hbm, sums_hbm, cnts_hbm,
             buf0, buf1, idsv, w1v, sums_v, cnts_v, acc_v, sem0, sem1):
    wid = lax.axis_index("s") * 2 + lax.axis_index("c")
    base = _SC_OFF + wid * _RPW

    pltpu.sync_copy(w_hbm.at[:, pl.ds(0, _C)], w1v)
    pltpu.sync_copy(ids_hbm.at[pl.ds(base, _RPW)], idsv.at[pl.ds(0, _RPW)])
    pltpu.async_copy(b_hbm.at[pl.ds(base, _CH), :], buf0, sem0)
    pltpu.async_copy(b_hbm.at[pl.ds(base + _CH, _CH), :], buf1, sem1)

    zf = jnp.zeros((16,), jnp.float32)
    zi = jnp.zeros((16,), jnp.int32)
    for gi in range(_G):
        sums_v[gi, pl.ds(0, 16)] = zf
        cnts_v[gi, pl.ds(0, 16)] = zi
    for j in range(_NSL):
        acc_v[j, pl.ds(0, 16)] = zf

    def flush(g, cnt):
        @pl.when(g >= 0)
        def _():
            v = acc_v[0, pl.ds(0, 16)] * w1v[0, pl.ds(0, 16)]
            for j in range(1, _NSL):
                v = v + acc_v[j, pl.ds(0, 16)] * w1v[0, pl.ds(j * 16, 16)]
            sums_v[g, pl.ds(0, 16)] = v
            cnts_v[g, pl.ds(0, 16)] = jnp.full((16,), cnt, jnp.int32)

    def make_group_body(buf, choff):
        def group_body(gq, carry):
            g = carry[0]
            rloc = gq * 16
            first = idsv[pl.ds(choff + rloc, 16)][0]
            last = idsv[pl.ds(choff + rloc + 15, 16)][0]
            same = (first == g) & (last == g)

            def fast(ops):
                for j in range(_NSL):
                    dsj = pl.ds(j * 16, 16)
                    a0 = buf[rloc, dsj] + buf[rloc + 1, dsj]
                    a1 = buf[rloc + 2, dsj] + buf[rloc + 3, dsj]
                    a2 = buf[rloc + 4, dsj] + buf[rloc + 5, dsj]
                    a3 = buf[rloc + 6, dsj] + buf[rloc + 7, dsj]
                    for k in range(8, 16, 4):
                        a0 = a0 + buf[rloc + k, dsj]
                        a1 = a1 + buf[rloc + k + 1, dsj]
                        a2 = a2 + buf[rloc + k + 2, dsj]
                        a3 = a3 + buf[rloc + k + 3, dsj]
                    acc_v[j, pl.ds(0, 16)] += (a0 + a1) + (a2 + a3)
                return (ops[0], ops[1] + 16)

            def slow(ops):
                def row_body(k, c2):
                    g1, cnt1 = c2
                    gr = idsv[pl.ds(choff + rloc + k, 16)][0]
                    changed = gr != g1
                    @pl.when(changed)
                    def _():
                        flush(g1, cnt1)
                        for j in range(_NSL):
                            acc_v[j, pl.ds(0, 16)] = zf
                    cnt1 = jnp.where(changed, 0, cnt1)
                    for j in range(_NSL):
                        acc_v[j, pl.ds(0, 16)] += buf[rloc + k,
                                                      pl.ds(j * 16, 16)]
                    return (gr, cnt1 + 1)
                return lax.fori_loop(0, 16, row_body, ops)

            return lax.cond(same, fast, slow, carry)
        return group_body

    def pair_body(p, carry):
        for bsel, (bufb, semb) in enumerate(((buf0, sem0), (buf1, sem1))):
            ch = p * 2 + bsel
            choff = ch * _CH
            pltpu.make_async_copy(b_hbm.at[pl.ds(0, _CH), :], bufb, semb).wait()
            carry = lax.fori_loop(0, _CH // 16,
                                  make_group_body(bufb, choff), carry)
            @pl.when(ch + 2 < _NCH)
            def _():
                nstart = base + (ch + 2) * _CH
                pltpu.async_copy(b_hbm.at[pl.ds(nstart, _CH), :], bufb, semb)
        return carry

    carry = lax.fori_loop(0, _NCH // 2, pair_body,
                          (jnp.int32(-1), jnp.int32(0)))
    flush(carry[0], carry[1])

    pltpu.sync_copy(sums_v, sums_hbm.at[wid])
    pltpu.sync_copy(cnts_v, cnts_hbm.at[wid])


def _sc_partials(B_z, ids_b, W):
    fn = functools.partial(
        pl.kernel,
        mesh=plsc.VectorSubcoreMesh(core_axis_name="c", subcore_axis_name="s"),
        out_type=[jax.ShapeDtypeStruct((_NW, _G, 16), jnp.float32),
                  jax.ShapeDtypeStruct((_NW, _G, 16), jnp.int32)],
        scratch_types=[pltpu.VMEM((_CH, _C), jnp.float32),
                       pltpu.VMEM((_CH, _C), jnp.float32),
                       pltpu.VMEM((_RPW + 16,), jnp.int32),
                       pltpu.VMEM((1, _C), jnp.float32),
                       pltpu.VMEM((_G, 16), jnp.float32),
                       pltpu.VMEM((_G, 16), jnp.int32),
                       pltpu.VMEM((_NSL, 16), jnp.float32),
                       pltpu.SemaphoreType.DMA,
                       pltpu.SemaphoreType.DMA],
    )(_sc_body)
    return fn(B_z, ids_b, W)


def _tc_main_body(ib_ref, ig_ref, bsh_ref, g_ref, w_ref, accb_ref, accg_ref):
    i = pl.program_id(0)

    @pl.when(i == 0)
    def _init():
        accb_ref[...] = jnp.zeros_like(accb_ref)
        accg_ref[...] = jnp.zeros_like(accg_ref)

    w1 = w_ref[0, :_C]
    w2 = w_ref[0, _C:]
    sv_b = jnp.sum(bsh_ref[...] * w1[None, :], axis=1, keepdims=True)
    sv_g = jnp.sum(g_ref[...] * w2[None, :], axis=1, keepdims=True)
    svc_b = jnp.concatenate([sv_b, jnp.ones_like(sv_b)], axis=1)
    svc_g = jnp.concatenate([sv_g, jnp.ones_like(sv_g)], axis=1)
    ids_b = ib_ref[0]
    ids_g = ig_ref[0]
    seg_b = lax.broadcasted_iota(jnp.int32, (_G, _RB), 0)
    seg_g = lax.broadcasted_iota(jnp.int32, (_G, _RG), 0)
    oh_b = (seg_b == ids_b).astype(jnp.float32)
    oh_g = (seg_g == ids_g).astype(jnp.float32)
    dn = (((1,), (0,)), ((), ()))
    accb_ref[...] += lax.dot_general(
        oh_b, svc_b, dn, preferred_element_type=jnp.float32)
    accg_ref[...] += lax.dot_general(
        oh_g, svc_g, dn, preferred_element_type=jnp.float32)


def _tc_main(ids_b_sh, ids_g, B_z, G_z, W):
    return pl.pallas_call(
        _tc_main_body,
        grid=(_NSTEPS,),
        in_specs=[
            pl.BlockSpec((1, 1, _RB), lambda i: (i, 0, 0)),
            pl.BlockSpec((1, 1, _RG), lambda i: (i, 0, 0)),
            pl.BlockSpec((_RB, _C), lambda i: (i, 0)),
            pl.BlockSpec((_RG, _C), lambda i: (i, 0)),
            pl.BlockSpec((1, 2 * _C), lambda i: (0, 0)),
        ],
        out_specs=[pl.BlockSpec((_G, 2), lambda i: (0, 0)),
                   pl.BlockSpec((_G, 2), lambda i: (0, 0))],
        out_shape=[jax.ShapeDtypeStruct((_G, 2), jnp.float32),
                   jax.ShapeDtypeStruct((_G, 2), jnp.float32)],
        compiler_params=pltpu.CompilerParams(
            dimension_semantics=("arbitrary",)),
    )(ids_b_sh, ids_g, B_z, G_z, W)


def _epi_body(scs_ref, scc_ref, accb_ref, accg_ref, bias_ref, out_ref):
    scs = jnp.sum(scs_ref[...], axis=(0, 2))
    scc = jnp.sum(scc_ref[...], axis=(0, 2)).astype(jnp.float32) / 16.0
    bsum = accb_ref[:, 0] + scs
    bcnt = accb_ref[:, 1] + scc
    res = (bsum / jnp.maximum(bcnt, 1.0)
           + accg_ref[:, 0] / jnp.maximum(accg_ref[:, 1], 1.0)
           + bias_ref[0, 0])
    out_ref[...] = res[:, None]


def _epilogue(sc_sums, sc_cnts, accb, accg, bias):
    return pl.pallas_call(
        _epi_body,
        out_shape=jax.ShapeDtypeStruct((_G, 1), jnp.float32),
    )(sc_sums, sc_cnts, accb, accg, bias)


def kernel(B_z, G_z, x_b_batch, x_g_batch, W, b):
    ids_b = x_b_batch.astype(jnp.int32)
    ids_g = x_g_batch.astype(jnp.int32)
    sc_sums, sc_cnts = _sc_partials(B_z, ids_b, W)
    ids_b_sh = ids_b[:_SC_OFF].reshape(_NSTEPS, 1, _RB)
    ids_g_r = ids_g.reshape(_NSTEPS, 1, _RG)
    accb, accg = _tc_main(ids_b_sh, ids_g_r, B_z, G_z, W)
    return _epilogue(sc_sums, sc_cnts, accb, accg, b.reshape(1, 1))


# speedup vs baseline: 1.6184x; 1.0010x over previous
import functools

import jax
import jax.numpy as jnp
from jax import lax
from jax.experimental import pallas as pl
from jax.experimental.pallas import tpu as pltpu
from jax.experimental.pallas import tpu_sc as plsc

_G = 128
_C = 256
_NSL = _C // 16

_SC_ROWS = 15360
_SC_OFF = 50000 - _SC_ROWS
_NW = 32
_RPW = _SC_ROWS // _NW
_CH = 80
_NCH = _RPW // _CH

_NSTEPS = 10
_RB = _SC_OFF // _NSTEPS
_RG = 50000 // _NSTEPS


def _sc_body(b_hbm, ids_hbm, w_hbm, out_hbm,
             buf0, buf1, idsv, w1v, sums_v, cnts_v, acc_v, sem0, sem1):
    wid = lax.axis_index("s") * 2 + lax.axis_index("c")
    base = _SC_OFF + wid * _RPW

    pltpu.sync_copy(w_hbm.at[:, pl.ds(0, _C)], w1v)
    pltpu.sync_copy(ids_hbm.at[pl.ds(base, _RPW)], idsv.at[pl.ds(0, _RPW)])
    pltpu.async_copy(b_hbm.at[pl.ds(base, _CH), :], buf0, sem0)
    pltpu.async_copy(b_hbm.at[pl.ds(base + _CH, _CH), :], buf1, sem1)

    zf = jnp.zeros((16,), jnp.float32)
    for gi in range(_G):
        sums_v[gi, pl.ds(0, 16)] = zf
        cnts_v[gi, pl.ds(0, 16)] = zf
    for j in range(_NSL):
        acc_v[j, pl.ds(0, 16)] = zf

    def flush(g, cnt):
        @pl.when(g >= 0)
        def _():
            v = acc_v[0, pl.ds(0, 16)] * w1v[0, pl.ds(0, 16)]
            for j in range(1, _NSL):
                v = v + acc_v[j, pl.ds(0, 16)] * w1v[0, pl.ds(j * 16, 16)]
            sums_v[g, pl.ds(0, 16)] = v
            cnts_v[g, pl.ds(0, 16)] = jnp.full((16,), 1.0,
                                               jnp.float32) * cnt.astype(
                                                   jnp.float32)

    def make_group_body(buf, choff):
        def group_body(gq, carry):
            g = carry[0]
            rloc = gq * 16
            first = idsv[pl.ds(choff + rloc, 16)][0]
            last = idsv[pl.ds(choff + rloc + 15, 16)][0]
            same = (first == g) & (last == g)

            def fast(ops):
                for j in range(_NSL):
                    dsj = pl.ds(j * 16, 16)
                    a0 = buf[rloc, dsj] + buf[rloc + 1, dsj]
                    a1 = buf[rloc + 2, dsj] + buf[rloc + 3, dsj]
                    a2 = buf[rloc + 4, dsj] + buf[rloc + 5, dsj]
                    a3 = buf[rloc + 6, dsj] + buf[rloc + 7, dsj]
                    for k in range(8, 16, 4):
                        a0 = a0 + buf[rloc + k, dsj]
                        a1 = a1 + buf[rloc + k + 1, dsj]
                        a2 = a2 + buf[rloc + k + 2, dsj]
                        a3 = a3 + buf[rloc + k + 3, dsj]
                    acc_v[j, pl.ds(0, 16)] += (a0 + a1) + (a2 + a3)
                return (ops[0], ops[1] + 16)

            def slow(ops):
                def row_body(k, c2):
                    g1, cnt1 = c2
                    gr = idsv[pl.ds(choff + rloc + k, 16)][0]
                    changed = gr != g1
                    @pl.when(changed)
                    def _():
                        flush(g1, cnt1)
                        for j in range(_NSL):
                            acc_v[j, pl.ds(0, 16)] = zf
                    cnt1 = jnp.where(changed, 0, cnt1)
                    for j in range(_NSL):
                        acc_v[j, pl.ds(0, 16)] += buf[rloc + k,
                                                      pl.ds(j * 16, 16)]
                    return (gr, cnt1 + 1)
                return lax.fori_loop(0, 16, row_body, ops)

            return lax.cond(same, fast, slow, carry)
        return group_body

    def pair_body(p, carry):
        for bsel, (bufb, semb) in enumerate(((buf0, sem0), (buf1, sem1))):
            ch = p * 2 + bsel
            choff = ch * _CH
            pltpu.make_async_copy(b_hbm.at[pl.ds(0, _CH), :], bufb, semb).wait()
            carry = lax.fori_loop(0, _CH // 16,
                                  make_group_body(bufb, choff), carry)
            @pl.when(ch + 2 < _NCH)
            def _():
                nstart = base + (ch + 2) * _CH
                pltpu.async_copy(b_hbm.at[pl.ds(nstart, _CH), :], bufb, semb)
        return carry

    carry = lax.fori_loop(0, _NCH // 2, pair_body,
                          (jnp.int32(-1), jnp.int32(0)))
    flush(carry[0], carry[1])

    pltpu.sync_copy(sums_v, out_hbm.at[wid, 0])
    pltpu.sync_copy(cnts_v, out_hbm.at[wid, 1])


def _sc_partials(B_z, ids_b, W):
    fn = functools.partial(
        pl.kernel,
        mesh=plsc.VectorSubcoreMesh(core_axis_name="c", subcore_axis_name="s"),
        out_type=jax.ShapeDtypeStruct((_NW, 2, _G, 16), jnp.float32),
        scratch_types=[pltpu.VMEM((_CH, _C), jnp.float32),
                       pltpu.VMEM((_CH, _C), jnp.float32),
                       pltpu.VMEM((_RPW + 16,), jnp.int32),
                       pltpu.VMEM((1, _C), jnp.float32),
                       pltpu.VMEM((_G, 16), jnp.float32),
                       pltpu.VMEM((_G, 16), jnp.float32),
                       pltpu.VMEM((_NSL, 16), jnp.float32),
                       pltpu.SemaphoreType.DMA,
                       pltpu.SemaphoreType.DMA],
    )(_sc_body)
    return fn(B_z, ids_b, W)


def _tc_main_body(ib_ref, ig_ref, bsh_ref, g_ref, w_ref, accb_ref, accg_ref):
    i = pl.program_id(0)

    @pl.when(i == 0)
    def _init():
        accb_ref[...] = jnp.zeros_like(accb_ref)
        accg_ref[...] = jnp.zeros_like(accg_ref)

    w1 = w_ref[0, :_C]
    w2 = w_ref[0, _C:]
    sv_b = jnp.sum(bsh_ref[...] * w1[None, :], axis=1, keepdims=True)
    sv_g = jnp.sum(g_ref[...] * w2[None, :], axis=1, keepdims=True)
    svc_b = jnp.concatenate([sv_b, jnp.ones_like(sv_b)], axis=1)
    svc_g = jnp.concatenate([sv_g, jnp.ones_like(sv_g)], axis=1)
    ids_b = ib_ref[0]
    ids_g = ig_ref[0]
    seg_b = lax.broadcasted_iota(jnp.int32, (_G, _RB), 0)
    seg_g = lax.broadcasted_iota(jnp.int32, (_G, _RG), 0)
    oh_b = (seg_b == ids_b).astype(jnp.float32)
    oh_g = (seg_g == ids_g).astype(jnp.float32)
    dn = (((1,), (0,)), ((), ()))
    accb_ref[...] += lax.dot_general(
        oh_b, svc_b, dn, preferred_element_type=jnp.float32)
    accg_ref[...] += lax.dot_general(
        oh_g, svc_g, dn, preferred_element_type=jnp.float32)


def _tc_main(ids_b_sh, ids_g, B_z, G_z, W):
    return pl.pallas_call(
        _tc_main_body,
        grid=(_NSTEPS,),
        in_specs=[
            pl.BlockSpec((1, 1, _RB), lambda i: (i, 0, 0)),
            pl.BlockSpec((1, 1, _RG), lambda i: (i, 0, 0)),
            pl.BlockSpec((_RB, _C), lambda i: (i, 0)),
            pl.BlockSpec((_RG, _C), lambda i: (i, 0)),
            pl.BlockSpec((1, 2 * _C), lambda i: (0, 0)),
        ],
        out_specs=[pl.BlockSpec((_G, 2), lambda i: (0, 0)),
                   pl.BlockSpec((_G, 2), lambda i: (0, 0))],
        out_shape=[jax.ShapeDtypeStruct((_G, 2), jnp.float32),
                   jax.ShapeDtypeStruct((_G, 2), jnp.float32)],
        compiler_params=pltpu.CompilerParams(
            dimension_semantics=("arbitrary",)),
    )(ids_b_sh, ids_g, B_z, G_z, W)


def _epi_body(sc_ref, accb_ref, accg_ref, bias_ref, out_ref):
    scs = jnp.sum(sc_ref[:, 0], axis=(0, 2))
    scc = jnp.sum(sc_ref[:, 1], axis=(0, 2)) / 16.0
    bsum = accb_ref[:, 0] + scs
    bcnt = accb_ref[:, 1] + scc
    res = (bsum / jnp.maximum(bcnt, 1.0)
           + accg_ref[:, 0] / jnp.maximum(accg_ref[:, 1], 1.0)
           + bias_ref[0, 0])
    out_ref[...] = res[:, None]


def _epilogue(sc_part, accb, accg, bias):
    return pl.pallas_call(
        _epi_body,
        out_shape=jax.ShapeDtypeStruct((_G, 1), jnp.float32),
    )(sc_part, accb, accg, bias)


def kernel(B_z, G_z, x_b_batch, x_g_batch, W, b):
    ids_b = x_b_batch.astype(jnp.int32)
    ids_g = x_g_batch.astype(jnp.int32)
    sc_part = _sc_partials(B_z, ids_b, W)
    ids_b_sh = ids_b[:_SC_OFF].reshape(_NSTEPS, 1, _RB)
    ids_g_r = ids_g.reshape(_NSTEPS, 1, _RG)
    accb, accg = _tc_main(ids_b_sh, ids_g_r, B_z, G_z, W)
    return _epilogue(sc_part, accb, accg, b.reshape(1, 1))
